# baseline (speedup 1.0000x reference)
import functools

import jax
import jax.numpy as jnp
from jax import lax
from jax.experimental import pallas as pl
from jax.experimental.pallas import tpu as pltpu
from jax.experimental.pallas import tpu_sc as plsc

_K = 8
_NEG_INF = float("-inf")

_LSC = 9216
_LC = 256
_TC_LBLK = 1024


def _tc_body(mixed_ref, panel_ref, vals_ref, idx_ref):
    x = panel_ref[0] * mixed_ref[0]
    n = x.shape[0]
    iota = jax.lax.broadcasted_iota(jnp.int32, x.shape, 0)
    for k in range(_K):
        m = jnp.max(x, axis=0, keepdims=True)
        amax = jnp.argmax(x, axis=0)
        vals_ref[0, k, :] = m[0]
        if k == 0:
            first = jnp.min(jnp.where(x == m, iota, n), axis=0)
            idx_ref[0, 0, :] = first.astype(jnp.int32)
        if k + 1 < _K:
            x = jnp.where(iota == amax[None, :], _NEG_INF, x)


def _tc_topk(mixed, panel, lsc, lblk):
    p, n, l = panel.shape
    b = mixed.shape[0]
    ltc = l - lsc
    off = lsc // lblk
    mixed3 = mixed.reshape(b, 1, l)
    grid = (p, ltc // lblk)
    vals, idx = pl.pallas_call(
        _tc_body,
        grid=grid,
        in_specs=[
            pl.BlockSpec((1, 1, lblk), lambda i, j: (i // 4, 0, j + off)),
            pl.BlockSpec((1, n, lblk), lambda i, j: (i, 0, j + off)),
        ],
        out_specs=[
            pl.BlockSpec((1, _K, lblk), lambda i, j: (i, 0, j)),
            pl.BlockSpec((1, 1, lblk), lambda i, j: (i, 0, j)),
        ],
        out_shape=[
            jax.ShapeDtypeStruct((p, _K, ltc), jnp.float32),
            jax.ShapeDtypeStruct((p, 1, ltc), jnp.int32),
        ],
    )(mixed3, panel)
    return vals, idx.reshape(p, ltc)


def _sc_topk(mixed, panel, lsc):
    p, n, l = panel.shape
    half = lsc // 2
    nchunks = half // _LC

    mesh = plsc.VectorSubcoreMesh(core_axis_name="c", subcore_axis_name="s")

    @functools.partial(
        pl.kernel,
        mesh=mesh,
        out_type=[
            jax.ShapeDtypeStruct((p, _K, lsc), jnp.float32),
            jax.ShapeDtypeStruct((p, lsc), jnp.int32),
        ],
        scratch_types=[
            pltpu.VMEM((2, n, _LC), jnp.float32),
            pltpu.VMEM((half,), jnp.float32),
            pltpu.VMEM((_K, _LC), jnp.float32),
            pltpu.VMEM((_LC,), jnp.int32),
            pltpu.SemaphoreType.DMA,
            pltpu.SemaphoreType.DMA,
        ],
    )
    def sc_kernel(mixed_hbm, panel_hbm, vals_hbm, idx_hbm,
                  pbuf2, mbuf, vbuf, ibuf, sem0, sem1):
        wid = lax.axis_index("s") * 2 + lax.axis_index("c")
        pair = wid // 2
        b = pair // 4
        base = (wid % 2) * half

        pltpu.sync_copy(mixed_hbm.at[b, pl.ds(base, half)], mbuf)

        def panel_src(ci):
            return panel_hbm.at[pair, :, pl.ds(base + ci * _LC, _LC)]

        pltpu.async_copy(panel_src(0), pbuf2.at[0], sem0)

        def per_chunk(ci, _):
            c0 = base + ci * _LC
            par = lax.rem(ci, 2)
            nxt = jnp.minimum(ci + 1, nchunks - 1)

            @pl.when(par == 1)
            def _():
                pltpu.async_copy(panel_src(nxt), pbuf2.at[0], sem0)

            @pl.when(par == 0)
            def _():
                pltpu.async_copy(panel_src(nxt), pbuf2.at[1], sem1)

            @pl.when(par == 0)
            def _():
                pltpu.make_async_copy(panel_src(ci), pbuf2.at[0], sem0).wait()

            @pl.when(par == 1)
            def _():
                pltpu.make_async_copy(panel_src(ci), pbuf2.at[1], sem1).wait()

            pbuf = pbuf2.at[par]

            def per_group(g, _):
                nw = 2
                sls = [pl.ds(g * (16 * nw) + 16 * t, 16) for t in range(nw)]
                moff = ci * _LC
                mvs = [pl.ds(moff + g * (16 * nw) + 16 * t, 16)
                       for t in range(nw)]
                mvs = [mbuf[s] for s in mvs]
                neg = jnp.full((16,), _NEG_INF, jnp.float32)
                zero = jnp.zeros((16,), jnp.int32)
                init = tuple(((neg,) * _K, zero) for _ in range(nw))

                def ladder(rs, i0, v, ni):
                    c = [v > rs[j] for j in range(_K - 1)]
                    out = [jnp.maximum(rs[0], v)]
                    for j in range(1, _K):
                        out.append(jnp.where(
                            c[j - 1], rs[j - 1], jnp.maximum(v, rs[j])))
                    nsplat = jnp.full((16,), ni, jnp.int32)
                    return tuple(out), jnp.where(c[0], nsplat, i0)

                def per_n(ni, carry):
                    vs = [pbuf[ni, sls[t]] * mvs[t] for t in range(nw)]
                    return tuple(
                        ladder(carry[t][0], carry[t][1], vs[t], ni)
                        for t in range(nw))

                fin = lax.fori_loop(0, n, per_n, init, unroll=32)
                for t in range(nw):
                    for j in range(_K):
                        vbuf[j, sls[t]] = fin[t][0][j]
                    ibuf[sls[t]] = fin[t][1]
                return 0

            lax.fori_loop(0, _LC // 32, per_group, 0)
            pltpu.sync_copy(vbuf, vals_hbm.at[pair, :, pl.ds(c0, _LC)])
            pltpu.sync_copy(ibuf, idx_hbm.at[pair, pl.ds(c0, _LC)])
            return 0

        lax.fori_loop(0, nchunks, per_chunk, 0)
        fpar = nchunks % 2
        if fpar == 0:
            pltpu.make_async_copy(
                panel_src(nchunks - 1), pbuf2.at[0], sem0).wait()
        else:
            pltpu.make_async_copy(
                panel_src(nchunks - 1), pbuf2.at[1], sem1).wait()

    return sc_kernel(mixed, panel)


@jax.jit
def _run(input_mixed, ref_panel):
    b, a, n, l = ref_panel.shape
    panel = ref_panel.reshape(b * a, n, l)

    parts = []
    if _LSC > 0:
        parts.append(_sc_topk(input_mixed, panel, _LSC))
    if _LSC < l:
        parts.append(_tc_topk(input_mixed, panel, _LSC, _TC_LBLK))

    if len(parts) == 1:
        vals, idx = parts[0]
    else:
        vals = jnp.concatenate([parts[0][0], parts[1][0]], axis=2)
        idx = jnp.concatenate([parts[0][1], parts[1][1]], axis=1)
    return vals.reshape(b, a, _K, l), idx.reshape(b, a, l)


def kernel(input_mixed, ref_panel):
    return _run(input_mixed, ref_panel)


# speedup vs baseline: 1.0679x; 1.0679x over previous
import functools

import jax
import jax.numpy as jnp
from jax import lax
from jax.experimental import pallas as pl
from jax.experimental.pallas import tpu as pltpu
from jax.experimental.pallas import tpu_sc as plsc

_K = 8
_NEG_INF = float("-inf")

_LSC = 7168
_LC = 256
_TC_LBLK = 1024


def _tc_body(mixed_ref, panel_ref, vals_ref, idx_ref):
    x = panel_ref[0] * mixed_ref[0]
    n = x.shape[0]
    iota = jax.lax.broadcasted_iota(jnp.int32, x.shape, 0)
    for k in range(_K):
        m = jnp.max(x, axis=0, keepdims=True)
        amax = jnp.argmax(x, axis=0)
        vals_ref[0, k, :] = m[0]
        if k == 0:
            first = jnp.min(jnp.where(x == m, iota, n), axis=0)
            idx_ref[0, 0, :] = first.astype(jnp.int32)
        if k + 1 < _K:
            x = jnp.where(iota == amax[None, :], _NEG_INF, x)


def _tc_topk(mixed, panel, lsc, lblk):
    p, n, l = panel.shape
    b = mixed.shape[0]
    ltc = l - lsc
    off = lsc // lblk
    mixed3 = mixed.reshape(b, 1, l)
    grid = (p, ltc // lblk)
    vals, idx = pl.pallas_call(
        _tc_body,
        grid=grid,
        in_specs=[
            pl.BlockSpec((1, 1, lblk), lambda i, j: (i // 4, 0, j + off)),
            pl.BlockSpec((1, n, lblk), lambda i, j: (i, 0, j + off)),
        ],
        out_specs=[
            pl.BlockSpec((1, _K, lblk), lambda i, j: (i, 0, j)),
            pl.BlockSpec((1, 1, lblk), lambda i, j: (i, 0, j)),
        ],
        out_shape=[
            jax.ShapeDtypeStruct((p, _K, ltc), jnp.float32),
            jax.ShapeDtypeStruct((p, 1, ltc), jnp.int32),
        ],
    )(mixed3, panel)
    return vals, idx.reshape(p, ltc)


def _sc_topk(mixed, panel, lsc):
    p, n, l = panel.shape
    half = lsc // 2
    nchunks = half // _LC

    mesh = plsc.VectorSubcoreMesh(core_axis_name="c", subcore_axis_name="s")

    @functools.partial(
        pl.kernel,
        mesh=mesh,
        out_type=[
            jax.ShapeDtypeStruct((p, _K, lsc), jnp.float32),
            jax.ShapeDtypeStruct((p, lsc), jnp.int32),
        ],
        scratch_types=[
            pltpu.VMEM((2, n, _LC), jnp.float32),
            pltpu.VMEM((half,), jnp.float32),
            pltpu.VMEM((_K, _LC), jnp.float32),
            pltpu.VMEM((_LC,), jnp.int32),
            pltpu.SemaphoreType.DMA,
            pltpu.SemaphoreType.DMA,
        ],
    )
    def sc_kernel(mixed_hbm, panel_hbm, vals_hbm, idx_hbm,
                  pbuf2, mbuf, vbuf, ibuf, sem0, sem1):
        wid = lax.axis_index("s") * 2 + lax.axis_index("c")
        pair = wid // 2
        b = pair // 4
        base = (wid % 2) * half

        pltpu.sync_copy(mixed_hbm.at[b, pl.ds(base, half)], mbuf)

        def panel_src(ci):
            return panel_hbm.at[pair, :, pl.ds(base + ci * _LC, _LC)]

        pltpu.async_copy(panel_src(0), pbuf2.at[0], sem0)

        def per_chunk(ci, _):
            c0 = base + ci * _LC
            par = lax.rem(ci, 2)
            nxt = jnp.minimum(ci + 1, nchunks - 1)

            @pl.when(par == 1)
            def _():
                pltpu.async_copy(panel_src(nxt), pbuf2.at[0], sem0)

            @pl.when(par == 0)
            def _():
                pltpu.async_copy(panel_src(nxt), pbuf2.at[1], sem1)

            @pl.when(par == 0)
            def _():
                pltpu.make_async_copy(panel_src(ci), pbuf2.at[0], sem0).wait()

            @pl.when(par == 1)
            def _():
                pltpu.make_async_copy(panel_src(ci), pbuf2.at[1], sem1).wait()

            pbuf = pbuf2.at[par]

            def per_group(g, _):
                nw = 2
                sls = [pl.ds(g * (16 * nw) + 16 * t, 16) for t in range(nw)]
                moff = ci * _LC
                mvs = [pl.ds(moff + g * (16 * nw) + 16 * t, 16)
                       for t in range(nw)]
                mvs = [mbuf[s] for s in mvs]
                neg = jnp.full((16,), _NEG_INF, jnp.float32)
                zero = jnp.zeros((16,), jnp.int32)
                init = tuple(((neg,) * _K, zero) for _ in range(nw))

                def ladder(rs, i0, v, ni):
                    c = [v > rs[j] for j in range(_K - 1)]
                    out = [jnp.maximum(rs[0], v)]
                    for j in range(1, _K):
                        out.append(jnp.where(
                            c[j - 1], rs[j - 1], jnp.maximum(v, rs[j])))
                    nsplat = jnp.full((16,), ni, jnp.int32)
                    return tuple(out), jnp.where(c[0], nsplat, i0)

                def per_n(ni, carry):
                    vs = [pbuf[ni, sls[t]] * mvs[t] for t in range(nw)]
                    return tuple(
                        ladder(carry[t][0], carry[t][1], vs[t], ni)
                        for t in range(nw))

                fin = lax.fori_loop(0, n, per_n, init, unroll=32)
                for t in range(nw):
                    for j in range(_K):
                        vbuf[j, sls[t]] = fin[t][0][j]
                    ibuf[sls[t]] = fin[t][1]
                return 0

            lax.fori_loop(0, _LC // 32, per_group, 0)
            pltpu.sync_copy(vbuf, vals_hbm.at[pair, :, pl.ds(c0, _LC)])
            pltpu.sync_copy(ibuf, idx_hbm.at[pair, pl.ds(c0, _LC)])
            return 0

        lax.fori_loop(0, nchunks, per_chunk, 0)
        fpar = nchunks % 2
        if fpar == 0:
            pltpu.make_async_copy(
                panel_src(nchunks - 1), pbuf2.at[0], sem0).wait()
        else:
            pltpu.make_async_copy(
                panel_src(nchunks - 1), pbuf2.at[1], sem1).wait()

    return sc_kernel(mixed, panel)


@jax.jit
def _run(input_mixed, ref_panel):
    b, a, n, l = ref_panel.shape
    panel = ref_panel.reshape(b * a, n, l)

    parts = []
    if _LSC > 0:
        parts.append(_sc_topk(input_mixed, panel, _LSC))
    if _LSC < l:
        parts.append(_tc_topk(input_mixed, panel, _LSC, _TC_LBLK))

    if len(parts) == 1:
        vals, idx = parts[0]
    else:
        vals = jnp.concatenate([parts[0][0], parts[1][0]], axis=2)
        idx = jnp.concatenate([parts[0][1], parts[1][1]], axis=1)
    return vals.reshape(b, a, _K, l), idx.reshape(b, a, l)


def kernel(input_mixed, ref_panel):
    return _run(input_mixed, ref_panel)
